# Initial kernel scaffold; baseline (speedup 1.0000x reference)
import jax
import jax.numpy as jnp
from jax.experimental import pallas as pl


def kernel(x, adj_t, adj_t2):
    raise NotImplementedError("write your pallas kernel here")


# speedup vs baseline: 6.9086x; 6.9086x over previous
import functools

import jax
import jax.numpy as jnp
from jax import lax
from jax.experimental import pallas as pl
from jax.experimental.pallas import tpu as pltpu
from jax.experimental.pallas import tpu_sc as plsc

N_NODES = 10000
D_FEAT = 128
N_EDGES = 320000

NC = 2
NS = 16

EDGES_PER_TILE = N_EDGES // NS
CHUNK = 80
NCHUNK = EDGES_PER_TILE // CHUNK
IDXB = 25
NBLK = NCHUNK // IDXB
N_PAD = 10240
ROWS_PER_TILE = N_PAD // NS

_mesh = plsc.VectorSubcoreMesh(core_axis_name="c", subcore_axis_name="s")


@functools.partial(
    pl.kernel,
    mesh=_mesh,
    out_type=jax.ShapeDtypeStruct((NC, N_PAD, D_FEAT), jnp.float32),
    scratch_types=[
        pltpu.VMEM((IDXB, CHUNK), jnp.int32),
        pltpu.VMEM((IDXB, CHUNK), jnp.int32),
        pltpu.VMEM((CHUNK, D_FEAT), jnp.float32),
        pltpu.VMEM_SHARED((N_PAD, D_FEAT), jnp.float32),
        pltpu.SemaphoreType.DMA,
    ],
)
def _spmm2(x_hbm, srcs_hbm, dsts_hbm, zeros_hbm, out_hbm,
           src_v, dst_v, rows_v, acc_sh, sem):
    c = lax.axis_index("c")
    s = lax.axis_index("s")

    row0 = s * ROWS_PER_TILE
    pltpu.sync_copy(zeros_hbm.at[pl.ds(row0, ROWS_PER_TILE)],
                    acc_sh.at[pl.ds(row0, ROWS_PER_TILE)])
    plsc.subcore_barrier()

    def outer(b, carry):
        pltpu.sync_copy(srcs_hbm.at[c, s, b], src_v)
        pltpu.sync_copy(dsts_hbm.at[c, s, b], dst_v)

        def body(j, inner_carry):
            pltpu.async_copy(x_hbm.at[src_v.at[j]], rows_v, sem).wait()
            pltpu.sync_copy(rows_v, acc_sh.at[dst_v.at[j]], add=True)
            return inner_carry

        lax.fori_loop(0, IDXB, body, carry)
        return carry

    lax.fori_loop(0, NBLK, outer, 0)

    plsc.subcore_barrier()
    pltpu.sync_copy(acc_sh.at[pl.ds(row0, ROWS_PER_TILE)],
                    out_hbm.at[c, pl.ds(row0, ROWS_PER_TILE)])


def kernel(x, adj_t, adj_t2):
    srcs = jnp.stack([adj_t[1], adj_t2[1]]).reshape(NC, NS, NBLK, IDXB, CHUNK)
    dsts = jnp.stack([adj_t[0], adj_t2[0]]).reshape(NC, NS, NBLK, IDXB, CHUNK)
    zeros = jnp.zeros((N_PAD, D_FEAT), jnp.float32)
    out = _spmm2(x, srcs, dsts, zeros)
    return jnp.concatenate([out[0, :N_NODES], out[1, :N_NODES]], axis=1)


# speedup vs baseline: 10.8533x; 1.5710x over previous
import functools

import jax
import jax.numpy as jnp
from jax import lax
from jax.experimental import pallas as pl
from jax.experimental.pallas import tpu as pltpu
from jax.experimental.pallas import tpu_sc as plsc

N_NODES = 10000
D_FEAT = 128
N_EDGES = 320000

NC = 2
NS = 16

EDGES_PER_TILE = N_EDGES // NS
CHUNK = 80
NCHUNK = EDGES_PER_TILE // CHUNK
IDXB = 25
NBLK = NCHUNK // IDXB
ROWS_MAIN = 624
TAIL0 = NS * ROWS_MAIN
TAIL = N_NODES - TAIL0

_mesh = plsc.VectorSubcoreMesh(core_axis_name="c", subcore_axis_name="s")


@functools.partial(
    pl.kernel,
    mesh=_mesh,
    out_type=jax.ShapeDtypeStruct((NC, N_NODES, D_FEAT), jnp.float32),
    scratch_types=[
        pltpu.VMEM((IDXB, CHUNK), jnp.int32),
        pltpu.VMEM((IDXB, CHUNK), jnp.int32),
        pltpu.VMEM((CHUNK, D_FEAT), jnp.float32),
        pltpu.VMEM((CHUNK, D_FEAT), jnp.float32),
        pltpu.VMEM_SHARED((N_NODES, D_FEAT), jnp.float32),
        pltpu.SemaphoreType.DMA,
        pltpu.SemaphoreType.DMA,
    ],
)
def _spmm2(x_hbm, srcs_hbm, dsts_hbm, zeros_hbm, out_hbm,
           src_v, dst_v, rows0_v, rows1_v, acc_sh, sem0, sem1):
    c = lax.axis_index("c")
    s = lax.axis_index("s")

    row0 = s * ROWS_MAIN
    pltpu.sync_copy(zeros_hbm.at[pl.ds(row0, ROWS_MAIN)],
                    acc_sh.at[pl.ds(row0, ROWS_MAIN)])

    @pl.when(s == NS - 1)
    def _zero_tail():
        pltpu.sync_copy(zeros_hbm.at[pl.ds(TAIL0, TAIL)],
                        acc_sh.at[pl.ds(TAIL0, TAIL)])

    plsc.subcore_barrier()

    bufs = (rows0_v, rows1_v)
    sems = (sem0, sem1)

    def block(b, carry):
        pltpu.sync_copy(srcs_hbm.at[c, s, b], src_v)
        pltpu.sync_copy(dsts_hbm.at[c, s, b], dst_v)

        cp0 = pltpu.async_copy(x_hbm.at[src_v.at[0]], rows0_v, sem0)
        cp1 = pltpu.async_copy(x_hbm.at[src_v.at[1]], rows1_v, sem1)
        del cp0, cp1
        for j in range(IDXB):
            buf = bufs[j % 2]
            sem = sems[j % 2]
            pltpu.make_async_copy(x_hbm.at[src_v.at[j]], buf, sem).wait()
            pltpu.sync_copy(buf, acc_sh.at[dst_v.at[j]], add=True)
            if j + 2 < IDXB:
                pltpu.async_copy(x_hbm.at[src_v.at[j + 2]], buf, sem)
        return carry

    lax.fori_loop(0, NBLK, block, 0)

    plsc.subcore_barrier()
    pltpu.sync_copy(acc_sh.at[pl.ds(row0, ROWS_MAIN)],
                    out_hbm.at[c, pl.ds(row0, ROWS_MAIN)])

    @pl.when(s == NS - 1)
    def _out_tail():
        pltpu.sync_copy(acc_sh.at[pl.ds(TAIL0, TAIL)],
                        out_hbm.at[c, pl.ds(TAIL0, TAIL)])


def kernel(x, adj_t, adj_t2):
    srcs = jnp.stack([adj_t[1], adj_t2[1]]).reshape(NC, NS, NBLK, IDXB, CHUNK)
    dsts = jnp.stack([adj_t[0], adj_t2[0]]).reshape(NC, NS, NBLK, IDXB, CHUNK)
    zeros = jnp.zeros((N_NODES, D_FEAT), jnp.float32)
    out = _spmm2(x, srcs, dsts, zeros)
    return jnp.concatenate([out[0], out[1]], axis=1)
